# baseline (speedup 1.0000x reference)
import functools

import jax
import jax.numpy as jnp
from jax import lax
from jax.experimental import pallas as pl
from jax.experimental.pallas import tpu as pltpu
from jax.experimental.pallas import tpu_sc as plsc

VOCAB = 1000000
D = 64
NI = 16384
NJ = 50
NC = 2
NS = 16
NW = NC * NS
NB2 = 3907
TLROWS = NB2 * 128
ITB_PER_W = NI // 128 // NW

_mesh = plsc.VectorSubcoreMesh(core_axis_name="c", subcore_axis_name="s")
_tc_tiled = pltpu.CompilerParams(
    use_tc_tiling_on_sc=True, needs_layout_passes=False
)


@functools.partial(
    pl.kernel,
    out_type=jax.ShapeDtypeStruct((TLROWS, 128), jnp.float32),
    mesh=_mesh,
    scratch_types=[
        pltpu.VMEM((4, D, 256), jnp.float32),
        pltpu.VMEM((2, 128, 128), jnp.float32),
        pltpu.SemaphoreType.DMA((4,)),
        pltpu.SemaphoreType.DMA((2,)),
    ],
    compiler_params=_tc_tiled,
)
def _repack(t2_hbm, tl_hbm, src_v, dst_v, rsem, wsem):
    wid = lax.axis_index("s") * NC + lax.axis_index("c")
    cnt = jnp.where(wid < 3, 123, 122)
    start = wid * 122 + jnp.minimum(wid, 3)
    lanes = lax.iota(jnp.int32, 16)
    perm = [(lanes + k) & 15 for k in range(16)]
    crow = [h * D + c0 + lanes for h in range(2) for c0 in range(0, D, 16)]

    def read(b, s):
        blk = start + b
        c0 = pl.multiple_of(blk * 256, 128)

        @pl.when(blk < NB2 - 1)
        def _():
            pltpu.async_copy(t2_hbm.at[:, pl.ds(c0, 256)], src_v.at[s], rsem.at[s])

        @pl.when(blk == NB2 - 1)
        def _():
            pltpu.async_copy(
                t2_hbm.at[:, pl.ds(c0, 128)],
                src_v.at[s, :, pl.ds(0, 128)],
                rsem.at[s],
            )

    def wait_read(b, s):
        blk = start + b

        @pl.when(blk < NB2 - 1)
        def _():
            pltpu.make_async_copy(
                t2_hbm.at[:, pl.ds(0, 256)], src_v.at[s], rsem.at[s]
            ).wait()

        @pl.when(blk == NB2 - 1)
        def _():
            pltpu.make_async_copy(
                t2_hbm.at[:, pl.ds(0, 128)],
                src_v.at[s, :, pl.ds(0, 128)],
                rsem.at[s],
            ).wait()

    def transpose(s, d):
        @pl.loop(0, 8)
        def _(lb):
            for k in range(16):
                col = lb * 16 + perm[k]
                for hc in range(8):
                    v = plsc.load_gather(
                        src_v.at[s], [crow[hc % 4], (hc // 4) * 128 + col]
                    )
                    plsc.store_scatter(dst_v.at[d], [col, crow[hc]], v)

    def write(b, d):
        row0 = pl.multiple_of((start + b) * 128, 8)
        pltpu.async_copy(dst_v.at[d], tl_hbm.at[pl.ds(row0, 128)], wsem.at[d])

    def wait_write(d):
        pltpu.make_async_copy(
            dst_v.at[d], tl_hbm.at[pl.ds(0, 128)], wsem.at[d]
        ).wait()

    for s in range(4):
        read(s, s)

    @pl.loop(0, (123 + 3) // 4)
    def _(g):
        def half(b, s):
            wait_read(b, s)

            @pl.when(b >= 2)
            def _():
                wait_write(s % 2)

            transpose(s, s % 2)
            write(b, s % 2)

            @pl.when(b + 4 < cnt)
            def _():
                read(b + 4, s)

        b0 = 4 * g
        for q in range(4):
            @pl.when(b0 + q < cnt)
            def _(q=q):
                half(b0 + q, q)

    wait_write(0)
    wait_write(1)


@functools.partial(
    pl.kernel,
    out_type=jax.ShapeDtypeStruct((NJ, D, NI), jnp.float32),
    mesh=_mesh,
    scratch_types=[
        pltpu.VMEM((7, 8, 128), jnp.int32),
        pltpu.VMEM((2, 128), jnp.int32),
        pltpu.VMEM((2, 128), jnp.int32),
        pltpu.VMEM((2, 128, 128), jnp.float32),
        pltpu.VMEM((2, D, 128), jnp.float32),
        pltpu.SemaphoreType.DMA((2,)),
        pltpu.SemaphoreType.DMA((2,)),
    ],
    compiler_params=_tc_tiled,
)
def _lookup(x2_hbm, tl_hbm, q_hbm, xv, kidx, psel, g_v, t_v, gsem, wsem):
    wid = lax.axis_index("s") * NC + lax.axis_index("c")
    lanes = lax.iota(jnp.int32, 16)
    perm = [(lanes + k) & 15 for k in range(16)]
    rowv = [g * 16 + lanes for g in range(8)]

    @pl.loop(0, ITB_PER_W)
    def _(itb):
        it128 = pl.multiple_of((wid * ITB_PER_W + itb) * 128, 128)
        for jt in range(6):
            pltpu.sync_copy(
                x2_hbm.at[pl.ds(jt * 8, 8), pl.ds(it128, 128)], xv.at[jt]
            )
        pltpu.sync_copy(
            x2_hbm.at[pl.ds(48, 2), pl.ds(it128, 128)], xv.at[6, pl.ds(0, 2)]
        )

        def prep_and_gather(j, s):
            jt = j // 8
            jl = j % 8
            for g in range(8):
                xr = xv[jt, jl, pl.ds(g * 16, 16)]
                kidx[s, pl.ds(g * 16, 16)] = ((xr >> 8) << 7) | (xr & 127)
                psel[s, pl.ds(g * 16, 16)] = (xr >> 1) & D
            pltpu.async_copy(tl_hbm.at[kidx.at[s]], g_v.at[s], gsem.at[s])

        def wait_gather(s):
            pltpu.make_async_copy(
                tl_hbm.at[pl.ds(0, 128)], g_v.at[s], gsem.at[s]
            ).wait()

        def transpose(s):
            ps = [psel[s, pl.ds(g * 16, 16)] for g in range(8)]

            @pl.loop(0, 4)
            def _(cb):
                for k in range(16):
                    cperm = cb * 16 + perm[k]
                    for g in range(8):
                        v = plsc.load_gather(
                            g_v.at[s], [rowv[g], ps[g] + cperm]
                        )
                        plsc.store_scatter(t_v.at[s], [cperm, rowv[g]], v)

        def write(j, s):
            pltpu.async_copy(
                t_v.at[s], q_hbm.at[j, :, pl.ds(it128, 128)], wsem.at[s]
            )

        def wait_write(j, s):
            pltpu.make_async_copy(
                t_v.at[s], q_hbm.at[j, :, pl.ds(it128, 128)], wsem.at[s]
            ).wait()

        prep_and_gather(0, 0)
        prep_and_gather(1, 1)

        @pl.loop(0, NJ // 2)
        def _(g):
            def half(j, s):
                wait_gather(s)

                @pl.when(j >= 2)
                def _():
                    wait_write(j - 2, s)

                transpose(s)
                write(j, s)

                @pl.when(j + 2 < NJ)
                def _():
                    prep_and_gather(j + 2, s)

            half(2 * g, 0)
            half(2 * g + 1, 1)

        wait_write(NJ - 2, 0)
        wait_write(NJ - 1, 1)


def kernel(x, table):
    tl = _repack(table.T)
    q = _lookup(x.T, tl)
    return q.transpose(2, 0, 1)


# speedup vs baseline: 1.2012x; 1.2012x over previous
import functools

import jax
import jax.numpy as jnp
from jax import lax
from jax.experimental import pallas as pl
from jax.experimental.pallas import tpu as pltpu
from jax.experimental.pallas import tpu_sc as plsc

VOCAB = 1000000
D = 64
NI = 16384
NJ = 50
NC = 2
NS = 16
NW = NC * NS
NB2 = 3907
TLROWS = NB2 * 128
ITB_PER_W = NI // 128 // NW

_mesh = plsc.VectorSubcoreMesh(core_axis_name="c", subcore_axis_name="s")
_tc_tiled = pltpu.CompilerParams(
    use_tc_tiling_on_sc=True, needs_layout_passes=False
)


@functools.partial(
    pl.kernel,
    out_type=jax.ShapeDtypeStruct((TLROWS, 128), jnp.float32),
    mesh=_mesh,
    scratch_types=[
        pltpu.VMEM((4, D, 128), jnp.float32),
        pltpu.VMEM((4, D, 128), jnp.float32),
        pltpu.VMEM((2, 128, 128), jnp.float32),
        pltpu.SemaphoreType.DMA((4,)),
        pltpu.SemaphoreType.DMA((2,)),
    ],
    compiler_params=_tc_tiled,
)
def _repack(t2_hbm, tl_hbm, src0_v, src1_v, dst_v, rsem, wsem):
    wid = lax.axis_index("s") * NC + lax.axis_index("c")
    cnt = jnp.where(wid < 3, 123, 122)
    start = wid * 122 + jnp.minimum(wid, 3)
    lanes = lax.iota(jnp.int32, 16)
    perm = [(lanes + k) & 15 for k in range(16)]
    crow = [h * D + c0 + lanes for h in range(2) for c0 in range(0, D, 16)]

    def read(b, s):
        blk = start + b

        @pl.when(blk < NB2 - 1)
        def _():
            c1 = pl.multiple_of(blk * 256 + 128, 128)
            pltpu.async_copy(t2_hbm.at[:, pl.ds(c1, 128)], src1_v.at[s], rsem.at[s])

        c0 = pl.multiple_of(blk * 256, 128)
        pltpu.async_copy(t2_hbm.at[:, pl.ds(c0, 128)], src0_v.at[s], rsem.at[s])

    def wait_read(b, s):
        blk = start + b

        @pl.when(blk < NB2 - 1)
        def _():
            pltpu.make_async_copy(
                t2_hbm.at[:, pl.ds(0, 128)], src1_v.at[s], rsem.at[s]
            ).wait()

        pltpu.make_async_copy(
            t2_hbm.at[:, pl.ds(0, 128)], src0_v.at[s], rsem.at[s]
        ).wait()

    def transpose(s, d):
        @pl.loop(0, 8)
        def _(lb):
            for k in range(16):
                col = lb * 16 + perm[k]
                for hc in range(8):
                    src = src0_v if hc < 4 else src1_v
                    v = plsc.load_gather(src.at[s], [crow[hc % 4], col])
                    plsc.store_scatter(dst_v.at[d], [col, crow[hc]], v)

    def write(b, d):
        row0 = pl.multiple_of((start + b) * 128, 8)
        pltpu.async_copy(dst_v.at[d], tl_hbm.at[pl.ds(row0, 128)], wsem.at[d])

    def wait_write(d):
        pltpu.make_async_copy(
            dst_v.at[d], tl_hbm.at[pl.ds(0, 128)], wsem.at[d]
        ).wait()

    for s in range(4):
        read(s, s)

    @pl.loop(0, (123 + 3) // 4)
    def _(g):
        def half(b, s):
            wait_read(b, s)

            @pl.when(b >= 2)
            def _():
                wait_write(s % 2)

            transpose(s, s % 2)
            write(b, s % 2)

            @pl.when(b + 4 < cnt)
            def _():
                read(b + 4, s)

        b0 = 4 * g
        for q in range(4):
            @pl.when(b0 + q < cnt)
            def _(q=q):
                half(b0 + q, q)

    wait_write(0)
    wait_write(1)


@functools.partial(
    pl.kernel,
    out_type=jax.ShapeDtypeStruct((NJ, D, NI), jnp.float32),
    mesh=_mesh,
    scratch_types=[
        pltpu.VMEM((7, 8, 128), jnp.int32),
        pltpu.VMEM((2, 128), jnp.int32),
        pltpu.VMEM((2, 128), jnp.int32),
        pltpu.VMEM((2, 128, 128), jnp.float32),
        pltpu.VMEM((2, D, 128), jnp.float32),
        pltpu.SemaphoreType.DMA((2,)),
        pltpu.SemaphoreType.DMA((2,)),
    ],
    compiler_params=_tc_tiled,
)
def _lookup(x2_hbm, tl_hbm, q_hbm, xv, kidx, psel, g_v, t_v, gsem, wsem):
    wid = lax.axis_index("s") * NC + lax.axis_index("c")
    lanes = lax.iota(jnp.int32, 16)
    perm = [(lanes + k) & 15 for k in range(16)]
    rowv = [g * 16 + lanes for g in range(8)]

    @pl.loop(0, ITB_PER_W)
    def _(itb):
        it128 = pl.multiple_of((wid * ITB_PER_W + itb) * 128, 128)
        for jt in range(6):
            pltpu.sync_copy(
                x2_hbm.at[pl.ds(jt * 8, 8), pl.ds(it128, 128)], xv.at[jt]
            )
        pltpu.sync_copy(
            x2_hbm.at[pl.ds(48, 2), pl.ds(it128, 128)], xv.at[6, pl.ds(0, 2)]
        )

        def prep_and_gather(j, s):
            jt = j // 8
            jl = j % 8
            for g in range(8):
                xr = xv[jt, jl, pl.ds(g * 16, 16)]
                kidx[s, pl.ds(g * 16, 16)] = ((xr >> 8) << 7) | (xr & 127)
                psel[s, pl.ds(g * 16, 16)] = (xr >> 1) & D
            pltpu.async_copy(tl_hbm.at[kidx.at[s]], g_v.at[s], gsem.at[s])

        def wait_gather(s):
            pltpu.make_async_copy(
                tl_hbm.at[pl.ds(0, 128)], g_v.at[s], gsem.at[s]
            ).wait()

        def transpose(s):
            ps = [psel[s, pl.ds(g * 16, 16)] for g in range(8)]

            @pl.loop(0, 4)
            def _(cb):
                for k in range(16):
                    cperm = cb * 16 + perm[k]
                    for g in range(8):
                        v = plsc.load_gather(
                            g_v.at[s], [rowv[g], ps[g] + cperm]
                        )
                        plsc.store_scatter(t_v.at[s], [cperm, rowv[g]], v)

        def write(j, s):
            pltpu.async_copy(
                t_v.at[s], q_hbm.at[j, :, pl.ds(it128, 128)], wsem.at[s]
            )

        def wait_write(j, s):
            pltpu.make_async_copy(
                t_v.at[s], q_hbm.at[j, :, pl.ds(it128, 128)], wsem.at[s]
            ).wait()

        prep_and_gather(0, 0)
        prep_and_gather(1, 1)

        @pl.loop(0, NJ // 2)
        def _(g):
            def half(j, s):
                wait_gather(s)

                @pl.when(j >= 2)
                def _():
                    wait_write(j - 2, s)

                transpose(s)
                write(j, s)

                @pl.when(j + 2 < NJ)
                def _():
                    prep_and_gather(j + 2, s)

            half(2 * g, 0)
            half(2 * g + 1, 1)

        wait_write(NJ - 2, 0)
        wait_write(NJ - 1, 1)


def kernel(x, table):
    tl = _repack(table.T)
    q = _lookup(x.T, tl)
    return q.transpose(2, 0, 1)


# speedup vs baseline: 1.2401x; 1.0323x over previous
import functools

import jax
import jax.numpy as jnp
from jax import lax
from jax.experimental import pallas as pl
from jax.experimental.pallas import tpu as pltpu
from jax.experimental.pallas import tpu_sc as plsc

VOCAB = 1000000
D = 64
NI = 16384
NJ = 50
NC = 2
NS = 16
NW = NC * NS
NB2 = 3907
TLROWS = NB2 * 128
ITB_PER_W = NI // 128 // NW

_mesh = plsc.VectorSubcoreMesh(core_axis_name="c", subcore_axis_name="s")
_tc_tiled = pltpu.CompilerParams(
    use_tc_tiling_on_sc=True, needs_layout_passes=False
)


@functools.partial(
    pl.kernel,
    out_type=jax.ShapeDtypeStruct((TLROWS, 128), jnp.float32),
    mesh=_mesh,
    scratch_types=[
        pltpu.VMEM((2, D, 128), jnp.float32),
        pltpu.VMEM((2, D, 128), jnp.float32),
        pltpu.VMEM((2, 128, 128), jnp.float32),
        pltpu.SemaphoreType.DMA((2,)),
        pltpu.SemaphoreType.DMA((2,)),
    ],
    compiler_params=_tc_tiled,
)
def _repack(t2_hbm, tl_hbm, src0_v, src1_v, dst_v, rsem, wsem):
    wid = lax.axis_index("s") * NC + lax.axis_index("c")
    cnt = jnp.where(wid < 3, 123, 122)
    start = wid * 122 + jnp.minimum(wid, 3)
    lanes = lax.iota(jnp.int32, 16)
    perm = [(lanes + k) & 15 for k in range(16)]
    crow = [h * D + c0 + lanes for h in range(2) for c0 in range(0, D, 16)]

    def read(b, s):
        blk = start + b

        @pl.when(blk < NB2 - 1)
        def _():
            c1 = pl.multiple_of(blk * 256 + 128, 128)
            pltpu.async_copy(t2_hbm.at[:, pl.ds(c1, 128)], src1_v.at[s], rsem.at[s])

        c0 = pl.multiple_of(blk * 256, 128)
        pltpu.async_copy(t2_hbm.at[:, pl.ds(c0, 128)], src0_v.at[s], rsem.at[s])

    def wait_read(b, s):
        blk = start + b

        @pl.when(blk < NB2 - 1)
        def _():
            pltpu.make_async_copy(
                t2_hbm.at[:, pl.ds(0, 128)], src1_v.at[s], rsem.at[s]
            ).wait()

        pltpu.make_async_copy(
            t2_hbm.at[:, pl.ds(0, 128)], src0_v.at[s], rsem.at[s]
        ).wait()

    def transpose(s, d):
        @pl.loop(0, 8)
        def _(lb):
            for k in range(16):
                col = lb * 16 + perm[k]
                for hc in range(8):
                    src = src0_v if hc < 4 else src1_v
                    v = plsc.load_gather(src.at[s], [crow[hc % 4], col])
                    plsc.store_scatter(dst_v.at[d], [col, crow[hc]], v)

    def write(b, d):
        row0 = pl.multiple_of((start + b) * 128, 8)
        pltpu.async_copy(dst_v.at[d], tl_hbm.at[pl.ds(row0, 128)], wsem.at[d])

    def wait_write(d):
        pltpu.make_async_copy(
            dst_v.at[d], tl_hbm.at[pl.ds(0, 128)], wsem.at[d]
        ).wait()

    read(0, 0)
    read(1, 1)

    @pl.loop(0, (123 + 1) // 2)
    def _(g):
        def half(b, s):
            wait_read(b, s)

            @pl.when(b >= 2)
            def _():
                wait_write(s)

            transpose(s, s)
            write(b, s)

            @pl.when(b + 2 < cnt)
            def _():
                read(b + 2, s)

        b0 = 2 * g

        @pl.when(b0 < cnt)
        def _():
            half(b0, 0)

        @pl.when(b0 + 1 < cnt)
        def _():
            half(b0 + 1, 1)

    wait_write(0)
    wait_write(1)


@functools.partial(
    pl.kernel,
    out_type=jax.ShapeDtypeStruct((NJ, D, NI), jnp.float32),
    mesh=_mesh,
    scratch_types=[
        pltpu.VMEM((7, 8, 128), jnp.int32),
        pltpu.VMEM((2, 128), jnp.int32),
        pltpu.VMEM((2, 128), jnp.int32),
        pltpu.VMEM((2, 128, 128), jnp.float32),
        pltpu.VMEM((2, D, 128), jnp.float32),
        pltpu.SemaphoreType.DMA((2,)),
        pltpu.SemaphoreType.DMA((2,)),
    ],
    compiler_params=_tc_tiled,
)
def _lookup(x2_hbm, tl_hbm, q_hbm, xv, kidx, psel, g_v, t_v, gsem, wsem):
    wid = lax.axis_index("s") * NC + lax.axis_index("c")
    lanes = lax.iota(jnp.int32, 16)
    perm = [(lanes + k) & 15 for k in range(16)]
    rowv = [g * 16 + lanes for g in range(8)]

    @pl.loop(0, ITB_PER_W)
    def _(itb):
        it128 = pl.multiple_of((wid * ITB_PER_W + itb) * 128, 128)
        for jt in range(6):
            pltpu.sync_copy(
                x2_hbm.at[pl.ds(jt * 8, 8), pl.ds(it128, 128)], xv.at[jt]
            )
        pltpu.sync_copy(
            x2_hbm.at[pl.ds(48, 2), pl.ds(it128, 128)], xv.at[6, pl.ds(0, 2)]
        )

        def prep_and_gather(j, s):
            jt = j // 8
            jl = j % 8
            for g in range(8):
                xr = xv[jt, jl, pl.ds(g * 16, 16)]
                kidx[s, pl.ds(g * 16, 16)] = ((xr >> 8) << 7) | (xr & 127)
                psel[s, pl.ds(g * 16, 16)] = (xr >> 1) & D
            pltpu.async_copy(tl_hbm.at[kidx.at[s]], g_v.at[s], gsem.at[s])

        def wait_gather(s):
            pltpu.make_async_copy(
                tl_hbm.at[pl.ds(0, 128)], g_v.at[s], gsem.at[s]
            ).wait()

        def transpose(s):
            ps = [psel[s, pl.ds(g * 16, 16)] for g in range(8)]

            @pl.loop(0, 4)
            def _(cb):
                for k in range(16):
                    cperm = cb * 16 + perm[k]
                    for g in range(8):
                        v = plsc.load_gather(
                            g_v.at[s], [rowv[g], ps[g] + cperm]
                        )
                        plsc.store_scatter(t_v.at[s], [cperm, rowv[g]], v)

        def write(j, s):
            pltpu.async_copy(
                t_v.at[s], q_hbm.at[j, :, pl.ds(it128, 128)], wsem.at[s]
            )

        def wait_write(j, s):
            pltpu.make_async_copy(
                t_v.at[s], q_hbm.at[j, :, pl.ds(it128, 128)], wsem.at[s]
            ).wait()

        prep_and_gather(0, 0)
        prep_and_gather(1, 1)

        @pl.loop(0, NJ // 2)
        def _(g):
            def half(j, s):
                wait_gather(s)

                @pl.when(j >= 2)
                def _():
                    wait_write(j - 2, s)

                transpose(s)
                write(j, s)

                @pl.when(j + 2 < NJ)
                def _():
                    prep_and_gather(j + 2, s)

            half(2 * g, 0)
            half(2 * g + 1, 1)

        wait_write(NJ - 2, 0)
        wait_write(NJ - 1, 1)


def kernel(x, table):
    tl = _repack(table.T)
    q = _lookup(x.T, tl)
    return q.transpose(2, 0, 1)
